# baseline (speedup 1.0000x reference)
import functools
import math

import jax
import jax.numpy as jnp
from jax import lax
from jax.experimental import pallas as pl
from jax.experimental.pallas import tpu as pltpu
from jax.experimental.pallas import tpu_sc as plsc

D_MODEL = 32
BATCH = 16384
HIST = 50
SCALE = math.sqrt(D_MODEL)

_info = plsc.get_sparse_core_info()
NC = _info.num_cores
NS = _info.num_subcores
NW = NC * NS
B_PER_W = BATCH // NW
FLAT_ROWS = BATCH * HIST * D_MODEL // 128

_MESH = plsc.VectorSubcoreMesh(core_axis_name="c", subcore_axis_name="s")

NB = 16
N_CHUNKS = B_PER_W // NB
GROUPS_PER_B = 2 * HIST
FLAT_PER_CHUNK = NB * HIST * D_MODEL // 128


def _gather_body(w_hbm, x_hbm, outf_hbm, xb_v, rows_v, pack_v, sems):
    wid = lax.axis_index("s") * NC + lax.axis_index("c")
    b_base = wid * B_PER_W

    def stage_in(c, buf):
        b0 = b_base + c * NB
        pltpu.sync_copy(x_hbm.at[pl.ds(b0, NB), :], xb_v.at[buf])
        for i in range(NB):
            pltpu.async_copy(
                w_hbm.at[xb_v.at[buf, i, :]],
                rows_v.at[buf, i],
                sems.at[buf],
            )

    def drain(buf):
        for i in range(NB):
            pltpu.make_async_copy(
                w_hbm.at[xb_v.at[buf, i, :]],
                rows_v.at[buf, i],
                sems.at[buf],
            ).wait()

    def finish(c, buf):
        b0 = b_base + c * NB

        def pack_b(bi, carry2):
            for v in range(GROUPS_PER_B):
                g_row = (bi * GROUPS_PER_B + v) >> 3
                g_off = (((bi & 1) * 4 + v) & 7) * 16
                pack_v[g_row, pl.ds(g_off, 16)] = (
                    rows_v[buf, bi, v >> 1, pl.ds((v & 1) * 16, 16)] * SCALE
                )
            return carry2

        lax.fori_loop(0, NB, pack_b, 0)
        r0 = pl.multiple_of((b0 * HIST * D_MODEL) // 128, 8)
        pltpu.sync_copy(pack_v, outf_hbm.at[pl.ds(r0, FLAT_PER_CHUNK)])

    stage_in(0, 0)

    def pair_body(p, carry):
        c0 = 2 * p
        stage_in(c0 + 1, 1)
        drain(0)
        finish(c0, 0)

        @pl.when(c0 + 2 < N_CHUNKS)
        def _():
            stage_in(c0 + 2, 0)

        drain(1)
        finish(c0 + 1, 1)
        return carry

    lax.fori_loop(0, N_CHUNKS // 2, pair_body, 0)


_gather_kernel = functools.partial(
    pl.kernel,
    out_type=jax.ShapeDtypeStruct((FLAT_ROWS, 128), jnp.float32),
    mesh=_MESH,
    scratch_types=[
        pltpu.VMEM((2, NB, HIST), jnp.int32),
        pltpu.VMEM((2, NB, HIST, D_MODEL), jnp.float32),
        pltpu.VMEM((FLAT_PER_CHUNK, 128), jnp.float32),
        pltpu.SemaphoreType.DMA((2,)),
    ],
    compiler_params=pltpu.CompilerParams(use_tc_tiling_on_sc=False),
)(_gather_body)

NBF = 16
NF_CHUNKS = B_PER_W // NBF
FLAT_F = NBF * HIST * D_MODEL // 128
NBH = NBF // 2


def _format_body(flat_hbm, out_hbm, stage128_v, stage32_v):
    wid = lax.axis_index("s") * NC + lax.axis_index("c")
    b_base = wid * B_PER_W

    def chunk_body(c, carry):
        b0 = b_base + c * NBF
        r0 = pl.multiple_of((b0 * HIST * D_MODEL) // 128, 8)
        pltpu.sync_copy(flat_hbm.at[pl.ds(r0, FLAT_F)], stage128_v)
        for half in range(2):

            def repack_b(b, carry2, half=half):
                bg = half * NBH + b
                for v in range(GROUPS_PER_B):
                    kg_row = (bg * GROUPS_PER_B + v) >> 3
                    kg_off = (((b & 1) * 4 + v) & 7) * 16
                    stage32_v[b, v >> 1, pl.ds((v & 1) * 16, 16)] = (
                        stage128_v[kg_row, pl.ds(kg_off, 16)]
                    )
                return carry2

            lax.fori_loop(0, NBH, repack_b, 0)
            pltpu.sync_copy(
                stage32_v, out_hbm.at[pl.ds(b0 + half * NBH, NBH)]
            )
        return carry

    lax.fori_loop(0, NF_CHUNKS, chunk_body, 0)


_format_kernel = functools.partial(
    pl.kernel,
    out_type=jax.ShapeDtypeStruct((BATCH, HIST, D_MODEL), jnp.float32),
    mesh=_MESH,
    scratch_types=[
        pltpu.VMEM((FLAT_F, 128), jnp.float32),
        pltpu.VMEM((NBH, HIST, D_MODEL), jnp.float32),
    ],
    compiler_params=pltpu.CompilerParams(use_tc_tiling_on_sc=True),
)(_format_body)


@jax.jit
def kernel(x, weight):
    flat = _gather_kernel(weight, x)
    return _format_kernel(flat)


# speedup vs baseline: 1.0948x; 1.0948x over previous
import functools
import math

import jax
import jax.numpy as jnp
from jax import lax
from jax.experimental import pallas as pl
from jax.experimental.pallas import tpu as pltpu
from jax.experimental.pallas import tpu_sc as plsc

D_MODEL = 32
BATCH = 16384
HIST = 50
SCALE = math.sqrt(D_MODEL)

_info = plsc.get_sparse_core_info()
NC = _info.num_cores
NS = _info.num_subcores
NW = NC * NS
B_PER_W = BATCH // NW
NB = 16
N_CHUNKS = B_PER_W // NB


def _body(w_hbm, x_hbm, out_hbm, xb_v, rows_v, sems):
    wid = lax.axis_index("s") * NC + lax.axis_index("c")
    b_base = wid * B_PER_W

    def stage_in(c, buf):
        b0 = b_base + c * NB
        pltpu.sync_copy(x_hbm.at[pl.ds(b0, NB), :], xb_v.at[buf])
        for i in range(NB):
            pltpu.async_copy(
                w_hbm.at[xb_v.at[buf, i, :]],
                rows_v.at[buf, i],
                sems.at[buf],
            )

    def drain(buf):
        for i in range(NB):
            pltpu.make_async_copy(
                w_hbm.at[xb_v.at[buf, i, :]],
                rows_v.at[buf, i],
                sems.at[buf],
            ).wait()

    def finish(c, buf):
        b0 = b_base + c * NB
        pltpu.sync_copy(rows_v.at[buf], out_hbm.at[pl.ds(b0, NB)])

    stage_in(0, 0)

    def pair_body(p, carry):
        c0 = 2 * p
        stage_in(c0 + 1, 1)
        drain(0)
        finish(c0, 0)

        @pl.when(c0 + 2 < N_CHUNKS)
        def _():
            stage_in(c0 + 2, 0)

        drain(1)
        finish(c0 + 1, 1)
        return carry

    lax.fori_loop(0, N_CHUNKS // 2, pair_body, 0)


_sc_kernel = functools.partial(
    pl.kernel,
    out_type=jax.ShapeDtypeStruct((BATCH, HIST, D_MODEL), jnp.float32),
    mesh=plsc.VectorSubcoreMesh(core_axis_name="c", subcore_axis_name="s"),
    scratch_types=[
        pltpu.VMEM((2, NB, HIST), jnp.int32),
        pltpu.VMEM((2, NB, HIST, D_MODEL), jnp.float32),
        pltpu.SemaphoreType.DMA((2,)),
    ],
    compiler_params=pltpu.CompilerParams(use_tc_tiling_on_sc=False),
)(_body)


@jax.jit
def kernel(x, weight):
    return _sc_kernel(weight, x) * SCALE


# speedup vs baseline: 1.3619x; 1.2439x over previous
import functools
import math

import jax
import jax.numpy as jnp
from jax import lax
from jax.experimental import pallas as pl
from jax.experimental.pallas import tpu as pltpu
from jax.experimental.pallas import tpu_sc as plsc

D_MODEL = 32
BATCH = 16384
HIST = 50
SCALE = math.sqrt(D_MODEL)

_info = plsc.get_sparse_core_info()
NC = _info.num_cores
NS = _info.num_subcores
NW = NC * NS
B_PER_W = BATCH // NW
NB = 16
N_CHUNKS = B_PER_W // NB


def _body(w_hbm, x_hbm, out_hbm, xb_v, rows_v, sems):
    wid = lax.axis_index("s") * NC + lax.axis_index("c")
    b_base = wid * B_PER_W

    def stage_in(c, buf):
        b0 = b_base + c * NB
        pltpu.sync_copy(x_hbm.at[pl.ds(b0, NB), :], xb_v.at[buf])
        for i in range(NB):
            pltpu.async_copy(
                w_hbm.at[xb_v.at[buf, i, :]],
                rows_v.at[buf, i],
                sems.at[buf],
            )

    def drain(buf):
        for i in range(NB):
            pltpu.make_async_copy(
                w_hbm.at[xb_v.at[buf, i, :]],
                rows_v.at[buf, i],
                sems.at[buf],
            ).wait()

    def finish(c, buf):
        b0 = b_base + c * NB

        def scale_b(bi, carry2):
            def scale_k(k, carry3):
                rows_v[buf, bi, k >> 1, pl.ds((k & 1) * 16, 16)] = (
                    rows_v[buf, bi, k >> 1, pl.ds((k & 1) * 16, 16)] * SCALE
                )
                return carry3

            return lax.fori_loop(0, 2 * HIST, scale_k, carry2, unroll=4)

        lax.fori_loop(0, NB, scale_b, 0)
        pltpu.sync_copy(rows_v.at[buf], out_hbm.at[pl.ds(b0, NB)])

    stage_in(0, 0)

    def pair_body(p, carry):
        c0 = 2 * p
        stage_in(c0 + 1, 1)
        drain(0)
        finish(c0, 0)

        @pl.when(c0 + 2 < N_CHUNKS)
        def _():
            stage_in(c0 + 2, 0)

        drain(1)
        finish(c0 + 1, 1)
        return carry

    lax.fori_loop(0, N_CHUNKS // 2, pair_body, 0)


_sc_kernel = functools.partial(
    pl.kernel,
    out_type=jax.ShapeDtypeStruct((BATCH, HIST, D_MODEL), jnp.float32),
    mesh=plsc.VectorSubcoreMesh(core_axis_name="c", subcore_axis_name="s"),
    scratch_types=[
        pltpu.VMEM((2, NB, HIST), jnp.int32),
        pltpu.VMEM((2, NB, HIST, D_MODEL), jnp.float32),
        pltpu.SemaphoreType.DMA((2,)),
    ],
    compiler_params=pltpu.CompilerParams(use_tc_tiling_on_sc=False),
)(_body)


@jax.jit
def kernel(x, weight):
    return _sc_kernel(weight, x)
